# baseline (speedup 1.0000x reference)
import functools
import math

import jax
import jax.numpy as jnp
from jax import lax
from jax.experimental import pallas as pl
from jax.experimental.pallas import tpu as pltpu
from jax.experimental.pallas import tpu_sc as plsc

_B = 128
_V = 100000
_CHUNK = 20000
_NCHUNKS = _V // _CHUNK

_BETA = 0.7

_NC = 2
_NS = 16
_NW_USED = 8
_BPW = _B // _NW_USED


def _lse_body(l1_ref, l2_ref, out1_ref, out2_ref, m1, s1, m2, s2):
    pid = pl.program_id(0)

    @pl.when(pid == 0)
    def _init():
        neg_inf = jnp.full((1, _B), -jnp.inf, jnp.float32)
        zero = jnp.zeros((1, _B), jnp.float32)
        m1[...] = neg_inf
        m2[...] = neg_inf
        s1[...] = zero
        s2[...] = zero

    def _update(x, m_ref, s_ref):
        m_old = m_ref[...]
        m_new = jnp.maximum(m_old, jnp.max(x, axis=0, keepdims=True))
        s_ref[...] = s_ref[...] * jnp.exp(m_old - m_new) + jnp.sum(
            jnp.exp(x - m_new), axis=0, keepdims=True
        )
        m_ref[...] = m_new

    _update(l1_ref[...], m1, s1)
    _update(l2_ref[...], m2, s2)

    @pl.when(pid == _NCHUNKS - 1)
    def _finish():
        out1_ref[...] = m1[...] + jnp.log(s1[...])
        out2_ref[...] = m2[...] + jnp.log(s2[...])


def _lse_call(lt1, lt2):
    return pl.pallas_call(
        _lse_body,
        grid=(_NCHUNKS,),
        in_specs=[
            pl.BlockSpec((_CHUNK, _B), lambda i: (i, 0)),
            pl.BlockSpec((_CHUNK, _B), lambda i: (i, 0)),
        ],
        out_specs=[
            pl.BlockSpec((1, _B), lambda i: (0, 0)),
            pl.BlockSpec((1, _B), lambda i: (0, 0)),
        ],
        out_shape=[
            jax.ShapeDtypeStruct((1, _B), jnp.float32),
            jax.ShapeDtypeStruct((1, _B), jnp.float32),
        ],
        scratch_shapes=[pltpu.VMEM((1, _B), jnp.float32) for _ in range(4)],
    )(lt1, lt2)


@functools.partial(
    pl.kernel,
    out_type=[
        jax.ShapeDtypeStruct((_B,), jnp.float32),
        jax.ShapeDtypeStruct((_B,), jnp.float32),
    ],
    mesh=plsc.VectorSubcoreMesh(
        core_axis_name="c", subcore_axis_name="s", num_cores=_NC, num_subcores=_NS
    ),
    scratch_types=[
        pltpu.VMEM((_BPW,), jnp.int32),
        pltpu.VMEM((_BPW,), jnp.int32),
        pltpu.VMEM((_BPW,), jnp.float32),
        pltpu.VMEM((_BPW,), jnp.float32),
        pltpu.SemaphoreType.DMA,
        pltpu.SemaphoreType.DMA,
    ],
)
def _gather_kernel(
    lt1_hbm, lt2_hbm, val_hbm, g1_hbm, g2_hbm,
    val_v, idx_v, d1_v, d2_v, sem1, sem2,
):
    wid = lax.axis_index("s") * _NC + lax.axis_index("c")

    @pl.when(wid < _NW_USED)
    def _():
        base = wid * _BPW
        pltpu.sync_copy(val_hbm.at[pl.ds(base, _BPW)], val_v)
        lane = lax.iota(jnp.int32, _BPW)
        idx_v[...] = val_v[...] * _B + base + lane
        cp1 = pltpu.async_copy(lt1_hbm.at[idx_v], d1_v, sem1)
        cp2 = pltpu.async_copy(lt2_hbm.at[idx_v], d2_v, sem2)
        cp1.wait()
        cp2.wait()
        pltpu.sync_copy(d1_v, g1_hbm.at[pl.ds(base, _BPW)])
        pltpu.sync_copy(d2_v, g2_hbm.at[pl.ds(base, _BPW)])


@jax.jit
def kernel(logits_1, logits_2, value):
    lt1 = logits_1.T
    lt2 = logits_2.T
    val = value.astype(jnp.int32)

    lse1, lse2 = _lse_call(lt1, lt2)
    g1, g2 = _gather_kernel(lt1.reshape(-1), lt2.reshape(-1), val)

    lp1 = g1 - lse1[0] + math.log(_BETA)
    lp2 = g2 - lse2[0] + math.log(1.0 - _BETA)
    return jnp.logaddexp(lp1, lp2)[None, :]


# speedup vs baseline: 1.0834x; 1.0834x over previous
import math

import jax
import jax.numpy as jnp
from jax.experimental import pallas as pl
from jax.experimental.pallas import tpu as pltpu

_B = 128
_V = 100000
_CHUNK = 10000
_NCHUNKS = _V // _CHUNK

_BETA = 0.7


def _lse_kernel(l1_ref, l2_ref, val_ref, out_ref, m1, s1, g1, m2, s2, g2):
    pid = pl.program_id(0)

    @pl.when(pid == 0)
    def _init():
        neg_inf = jnp.full((1, _B), -jnp.inf, jnp.float32)
        zero = jnp.zeros((1, _B), jnp.float32)
        m1[...] = neg_inf
        m2[...] = neg_inf
        s1[...] = zero
        s2[...] = zero
        g1[...] = zero
        g2[...] = zero

    rows = pid * _CHUNK + jax.lax.broadcasted_iota(jnp.int32, (_CHUNK, _B), 0)
    eq = rows == val_ref[...]

    def _update(x, m_ref, s_ref, g_ref):
        m_old = m_ref[...]
        m_new = jnp.maximum(m_old, jnp.max(x, axis=0, keepdims=True))
        s_ref[...] = s_ref[...] * jnp.exp(m_old - m_new) + jnp.sum(
            jnp.exp(x - m_new), axis=0, keepdims=True
        )
        m_ref[...] = m_new
        g_ref[...] += jnp.sum(jnp.where(eq, x, 0.0), axis=0, keepdims=True)

    _update(l1_ref[...], m1, s1, g1)
    _update(l2_ref[...], m2, s2, g2)

    @pl.when(pid == _NCHUNKS - 1)
    def _finish():
        lp1 = g1[...] - m1[...] - jnp.log(s1[...]) + math.log(_BETA)
        lp2 = g2[...] - m2[...] - jnp.log(s2[...]) + math.log(1.0 - _BETA)
        mx = jnp.maximum(lp1, lp2)
        out_ref[...] = mx + jnp.log(jnp.exp(lp1 - mx) + jnp.exp(lp2 - mx))


@jax.jit
def kernel(logits_1, logits_2, value):
    lt1 = logits_1.T
    lt2 = logits_2.T
    val2d = value.astype(jnp.int32).reshape(1, _B)
    return pl.pallas_call(
        _lse_kernel,
        grid=(_NCHUNKS,),
        in_specs=[
            pl.BlockSpec((_CHUNK, _B), lambda i: (i, 0)),
            pl.BlockSpec((_CHUNK, _B), lambda i: (i, 0)),
            pl.BlockSpec((1, _B), lambda i: (0, 0)),
        ],
        out_specs=pl.BlockSpec((1, _B), lambda i: (0, 0)),
        out_shape=jax.ShapeDtypeStruct((1, _B), jnp.float32),
        scratch_shapes=[pltpu.VMEM((1, _B), jnp.float32) for _ in range(6)],
    )(lt1, lt2, val2d)


# speedup vs baseline: 1.3130x; 1.2119x over previous
import math

import jax
import jax.numpy as jnp
from jax.experimental import pallas as pl
from jax.experimental.pallas import tpu as pltpu

_B = 128
_V = 100000
_CHUNK = 10000
_NCHUNKS = _V // _CHUNK

_BETA = 0.7


def _lse_kernel(l1_ref, l2_ref, val_ref, out_ref, m1, s1, g1, m2, s2, g2):
    pid = pl.program_id(0)

    @pl.when(pid == 0)
    def _init():
        neg_inf = jnp.full((1, _B), -jnp.inf, jnp.float32)
        zero = jnp.zeros((1, _B), jnp.float32)
        m1[...] = neg_inf
        m2[...] = neg_inf
        s1[...] = zero
        s2[...] = zero
        g1[...] = zero
        g2[...] = zero

    def _update(x, m_ref, s_ref):
        m_old = m_ref[...]
        m_new = jnp.maximum(m_old, jnp.max(x, axis=0, keepdims=True))
        s_ref[...] = s_ref[...] * jnp.exp(m_old - m_new) + jnp.sum(
            jnp.exp(x - m_new), axis=0, keepdims=True
        )
        m_ref[...] = m_new

    _update(l1_ref[...], m1, s1)
    _update(l2_ref[...], m2, s2)

    lane = jax.lax.broadcasted_iota(jnp.int32, (1, _B), 1)
    acc1 = [jnp.zeros((1, _B), jnp.float32) for _ in range(4)]
    acc2 = [jnp.zeros((1, _B), jnp.float32) for _ in range(4)]
    for k in range(_B):
        vk = val_ref[0, k]
        row = vk - pid * _CHUNK
        inb = jnp.logical_and(row >= 0, row < _CHUNK)
        rowc = jnp.clip(row, 0, _CHUNK - 1)
        mask = jnp.logical_and(lane == k, inb)
        acc1[k % 4] += jnp.where(mask, l1_ref[pl.ds(rowc, 1), :], 0.0)
        acc2[k % 4] += jnp.where(mask, l2_ref[pl.ds(rowc, 1), :], 0.0)
    g1[...] += acc1[0] + acc1[1] + acc1[2] + acc1[3]
    g2[...] += acc2[0] + acc2[1] + acc2[2] + acc2[3]

    @pl.when(pid == _NCHUNKS - 1)
    def _finish():
        lp1 = g1[...] - m1[...] - jnp.log(s1[...]) + math.log(_BETA)
        lp2 = g2[...] - m2[...] - jnp.log(s2[...]) + math.log(1.0 - _BETA)
        mx = jnp.maximum(lp1, lp2)
        out_ref[...] = mx + jnp.log(jnp.exp(lp1 - mx) + jnp.exp(lp2 - mx))


@jax.jit
def kernel(logits_1, logits_2, value):
    lt1 = logits_1.T
    lt2 = logits_2.T
    val2d = value.astype(jnp.int32).reshape(1, _B)
    return pl.pallas_call(
        _lse_kernel,
        grid=(_NCHUNKS,),
        in_specs=[
            pl.BlockSpec((_CHUNK, _B), lambda i: (i, 0)),
            pl.BlockSpec((_CHUNK, _B), lambda i: (i, 0)),
            pl.BlockSpec(memory_space=pltpu.SMEM),
        ],
        out_specs=pl.BlockSpec((1, _B), lambda i: (0, 0)),
        out_shape=jax.ShapeDtypeStruct((1, _B), jnp.float32),
        scratch_shapes=[pltpu.VMEM((1, _B), jnp.float32) for _ in range(6)],
    )(lt1, lt2, val2d)
